# baseline (speedup 1.0000x reference)
import functools

import jax
import jax.numpy as jnp
from jax import lax
from jax.experimental import pallas as pl
from jax.experimental.pallas import tpu as pltpu
from jax.experimental.pallas import tpu_sc as plsc

_TB = 1024
_NC = 2
_NS = 16
_NW = _NC * _NS
_L = 16


def _pos_body(x_ref, wt8_ref, c2_ref, pos_ref, sel_ref):
    t = x_ref.shape[0]
    pos8 = jnp.dot(x_ref[...], wt8_ref[...],
                   preferred_element_type=jnp.float32)
    pos_ref[...] = pos8.T
    px = pos8[:, 0:1]
    py = pos8[:, 1:2]
    pz = pos8[:, 2:3]
    c2 = c2_ref[...]
    d = ((px - c2[0:1, :]) ** 2 + (py - c2[1:2, :]) ** 2
         + (pz - c2[2:3, :]) ** 2)
    it16 = lax.broadcasted_iota(jnp.int32, (t, 16), 1)
    inf = jnp.float32(jnp.inf)
    sels = []
    for _ in range(8):
        g = jnp.argmin(d, axis=1).astype(jnp.int32).reshape(t, 1)
        sels.append(g)
        d = jnp.where(it16 == g, inf, d)
    sel_ref[...] = jnp.concatenate(sels, axis=1).T


def _tc_pos(x, wt8, c2t):
    b, k = x.shape
    return pl.pallas_call(
        _pos_body,
        grid=(b // _TB,),
        in_specs=[
            pl.BlockSpec((_TB, k), lambda i: (i, 0)),
            pl.BlockSpec((k, 8), lambda i: (0, 0)),
            pl.BlockSpec((3, 16), lambda i: (0, 0)),
        ],
        out_specs=[
            pl.BlockSpec((8, _TB), lambda i: (0, i)),
            pl.BlockSpec((8, _TB), lambda i: (0, i)),
        ],
        out_shape=[
            jax.ShapeDtypeStruct((8, b), jnp.float32),
            jax.ShapeDtypeStruct((8, b), jnp.int32),
        ],
    )(x, wt8, c2t)


def _min_pick(ds, vals):
    ds, vals = list(ds), list(vals)
    while len(ds) > 1:
        nd, nv = [], []
        for a in range(0, len(ds), 2):
            le = ds[a] <= ds[a + 1]
            nd.append(jnp.minimum(ds[a], ds[a + 1]))
            nv.append(jnp.where(le, vals[a], vals[a + 1]))
        ds, vals = nd, nv
    return vals[0]


def _sc_route_body(tpw, posT, selT, c3f, out, px_v, py_v, pz_v, sel_v, c3_v,
                   out_v):
    wid = lax.axis_index("s") * _NC + lax.axis_index("c")
    base = wid * tpw
    pltpu.sync_copy(posT.at[0, pl.ds(base, tpw)], px_v)
    pltpu.sync_copy(posT.at[1, pl.ds(base, tpw)], py_v)
    pltpu.sync_copy(posT.at[2, pl.ds(base, tpw)], pz_v)
    for i in range(8):
        pltpu.sync_copy(selT.at[i, pl.ds(base, tpw)],
                        sel_v.at[pl.ds(i * tpw, tpw)])
    pltpu.sync_copy(c3f, c3_v)

    idx16 = lax.iota(jnp.int32, _L)
    inf = jnp.float32(jnp.inf)

    def group(g, carry):
        sl = pl.ds(g * _L, _L)
        px = px_v[sl]
        py = py_v[sl]
        pz = pz_v[sl]
        cands, d3 = [], []
        for i in range(8):
            gi4 = sel_v[pl.ds(i * tpw + g * _L, _L)] * 4
            for c in range(4):
                cand = gi4 + c
                gx = plsc.load_gather(c3_v, [cand])
                gy = plsc.load_gather(c3_v, [cand + 64])
                gz = plsc.load_gather(c3_v, [cand + 128])
                cands.append(cand)
                dx = px - gx
                dy = py - gy
                dz = pz - gz
                d3.append(dx * dx + dy * dy + dz * dz)
        for k in range(8):
            e = _min_pick(d3, cands)
            plsc.store_scatter(out_v, [idx16 * 8 + (g * (_L * 8) + k)], e)
            for j in range(32):
                d3[j] = jnp.where(e == cands[j], inf, d3[j])
        return carry

    lax.fori_loop(0, tpw // _L, group, 0)
    pltpu.sync_copy(out_v, out.at[pl.ds(base * 8, tpw * 8)])


def _sc_route(posT, selT, c3f):
    b = posT.shape[1]
    tpw = b // _NW
    mesh = plsc.VectorSubcoreMesh(core_axis_name="c", subcore_axis_name="s")
    return pl.kernel(
        functools.partial(_sc_route_body, tpw),
        mesh=mesh,
        compiler_params=pltpu.CompilerParams(needs_layout_passes=False),
        out_type=jax.ShapeDtypeStruct((b * 8,), jnp.int32),
        scratch_types=[
            pltpu.VMEM((tpw,), jnp.float32),
            pltpu.VMEM((tpw,), jnp.float32),
            pltpu.VMEM((tpw,), jnp.float32),
            pltpu.VMEM((tpw * 8,), jnp.int32),
            pltpu.VMEM((192,), jnp.float32),
            pltpu.VMEM((tpw * 8,), jnp.int32),
        ],
    )(posT, selT, c3f)


def kernel(x, W, l1_centers, l2_centers, l3_centers):
    del l1_centers
    b = x.shape[0]
    wt8 = jnp.zeros((W.shape[1], 8), jnp.float32).at[:, :3].set(W.T)
    c2t = l2_centers.reshape(16, 3).T
    c3f = l3_centers.reshape(64, 3).T.reshape(192)
    posT, selT = _tc_pos(x, wt8, c2t)
    outf = _sc_route(posT, selT, c3f)
    return outf.reshape(b, 8)


# speedup vs baseline: 1.3093x; 1.3093x over previous
import functools

import jax
import jax.numpy as jnp
from jax import lax
from jax.experimental import pallas as pl
from jax.experimental.pallas import tpu as pltpu
from jax.experimental.pallas import tpu_sc as plsc

_TB = 1024
_NC = 2
_NS = 16
_NW = _NC * _NS
_L = 16


def _pos_body(x_ref, wt8_ref, out_ref):
    pos8 = jnp.dot(x_ref[...], wt8_ref[...],
                   preferred_element_type=jnp.float32)
    out_ref[...] = pos8.T


def _tc_pos(x, wt8, nblk, blk0):
    k = x.shape[1]
    return pl.pallas_call(
        _pos_body,
        grid=(nblk,),
        in_specs=[
            pl.BlockSpec((_TB, k), lambda i: (i + blk0, 0)),
            pl.BlockSpec((k, 8), lambda i: (0, 0)),
        ],
        out_specs=pl.BlockSpec((8, _TB), lambda i: (0, i)),
        out_shape=jax.ShapeDtypeStruct((8, nblk * _TB), jnp.float32),
    )(x, wt8)


def _min_pick(ds, vals):
    ds, vals = list(ds), list(vals)
    while len(ds) > 1:
        nd, nv = [], []
        for a in range(0, len(ds), 2):
            le = ds[a] <= ds[a + 1]
            nd.append(jnp.minimum(ds[a], ds[a + 1]))
            nv.append(jnp.where(le, vals[a], vals[a + 1]))
        ds, vals = nd, nv
    return vals[0]


def _sc_route_body(tpw, posT, ctab, out, p3_v, ct_v, out_v):
    wid = lax.axis_index("s") * _NC + lax.axis_index("c")
    base = wid * tpw
    pltpu.sync_copy(posT.at[pl.ds(0, 3), pl.ds(base, tpw)], p3_v)
    pltpu.sync_copy(ctab, ct_v)

    idx16 = lax.iota(jnp.int32, _L)
    inf = jnp.float32(jnp.inf)
    c2x = ct_v[pl.ds(0, 16)]
    c2y = ct_v[pl.ds(16, 16)]
    c2z = ct_v[pl.ds(32, 16)]

    def group(g, carry):
        sl = pl.ds(g * _L, _L)
        px = p3_v[0, sl]
        py = p3_v[1, sl]
        pz = p3_v[2, sl]
        d2 = []
        for c in range(16):
            dx = px - c2x[c]
            dy = py - c2y[c]
            dz = pz - c2z[c]
            d2.append(dx * dx + dy * dy + dz * dz)
        ids16 = [jnp.full((_L,), c, jnp.int32) for c in range(16)]
        sels = []
        for _ in range(8):
            sel = _min_pick(d2, ids16)
            sels.append(sel)
            for c in range(16):
                d2[c] = jnp.where(sel == c, inf, d2[c])
        cands, d3 = [], []
        for i in range(8):
            gi4 = sels[i] * 4
            for c in range(4):
                cand = gi4 + c
                gx = plsc.load_gather(ct_v, [cand + 48])
                gy = plsc.load_gather(ct_v, [cand + 112])
                gz = plsc.load_gather(ct_v, [cand + 176])
                cands.append(cand)
                dx = px - gx
                dy = py - gy
                dz = pz - gz
                d3.append(dx * dx + dy * dy + dz * dz)
        for k in range(8):
            e = _min_pick(d3, cands)
            plsc.store_scatter(out_v, [idx16 * 8 + (g * (_L * 8) + k)], e)
            for j in range(32):
                d3[j] = jnp.where(e == cands[j], inf, d3[j])
        return carry

    lax.fori_loop(0, tpw // _L, group, 0)
    pltpu.sync_copy(out_v, out.at[pl.ds(base * 8, tpw * 8)])


def _sc_route(posT, ctab):
    b = posT.shape[1]
    tpw = b // _NW
    mesh = plsc.VectorSubcoreMesh(core_axis_name="c", subcore_axis_name="s")
    return pl.kernel(
        functools.partial(_sc_route_body, tpw),
        mesh=mesh,
        compiler_params=pltpu.CompilerParams(needs_layout_passes=False),
        out_type=jax.ShapeDtypeStruct((b * 8,), jnp.int32),
        scratch_types=[
            pltpu.VMEM((3, tpw), jnp.float32),
            pltpu.VMEM((240,), jnp.float32),
            pltpu.VMEM((tpw * 8,), jnp.int32),
        ],
    )(posT, ctab)


def kernel(x, W, l1_centers, l2_centers, l3_centers):
    del l1_centers
    b = x.shape[0]
    wt8 = jnp.zeros((W.shape[1], 8), jnp.float32).at[:, :3].set(W.T)
    ctab = jnp.concatenate([
        l2_centers.reshape(16, 3).T.reshape(48),
        l3_centers.reshape(64, 3).T.reshape(192),
    ])
    posT = _tc_pos(x, wt8, b // _TB, 0)
    outf = _sc_route(posT, ctab)
    return outf.reshape(b, 8)
